# Initial kernel scaffold; baseline (speedup 1.0000x reference)
import jax
import jax.numpy as jnp
from jax.experimental import pallas as pl


def kernel(in_embed, out_embed, edge_index_list):
    raise NotImplementedError("write your pallas kernel here")


# speedup vs baseline: 1.2015x; 1.2015x over previous
import functools

import jax
import jax.numpy as jnp
from jax import lax
from jax.experimental import pallas as pl
from jax.experimental.pallas import tpu as pltpu
from jax.experimental.pallas import tpu_sc as plsc

N_NODES = 100000
N_EDGES = 3200000
FEAT = 32
MAX_DEG = 64

NC = 2
NS = 16
NW = NC * NS
L = 16

IDX_W = 128
CHUNK_ROWS = 8
EDGES_PER_CHUNK = CHUNK_ROWS * IDX_W
N_EDGE_CHUNKS = N_EDGES // EDGES_PER_CHUNK
ROUNDS_A = -(-N_EDGE_CHUNKS // NW)

ZCHUNK = 2000
N_ZCHUNKS = N_NODES // ZCHUNK

NODE_CHUNK = 800
N_NODE_CHUNKS = N_NODES // NODE_CHUNK
ROUNDS_B = -(-N_NODE_CHUNKS // NW)
GROUPS = NODE_CHUNK // L

_mesh = plsc.VectorSubcoreMesh(
    core_axis_name="c", subcore_axis_name="s", num_cores=NC, num_subcores=NS
)


@functools.partial(
    pl.kernel,
    out_type=jax.ShapeDtypeStruct((NC * 2 * N_NODES,), jnp.int32),
    mesh=_mesh,
    scratch_types=[
        pltpu.VMEM((CHUNK_ROWS, IDX_W), jnp.int32),
        pltpu.VMEM((IDX_W,), jnp.int32),
        pltpu.VMEM((ZCHUNK,), jnp.int32),
        pltpu.VMEM_SHARED((N_NODES,), jnp.int32),
        pltpu.VMEM_SHARED((N_NODES,), jnp.int32),
    ],
)
def _count_kernel(edges_hbm, counts_hbm, idx_v, ones_v, zeros_v, deg_in_s, deg_out_s):
    cid = lax.axis_index("c")
    sid = lax.axis_index("s")
    wid = cid * NS + sid

    for j in range(IDX_W // L):
        ones_v[pl.ds(j * L, L)] = jnp.ones((L,), jnp.int32)

    def _zero_body(i, _):
        zeros_v[pl.ds(i * L, L)] = jnp.zeros((L,), jnp.int32)
        return 0

    lax.fori_loop(0, ZCHUNK // L, _zero_body, 0)

    for arr in (deg_in_s, deg_out_s):
        for k in range(-(-N_ZCHUNKS // NS)):
            z = sid + k * NS

            @pl.when(z < N_ZCHUNKS)
            def _():
                pltpu.sync_copy(zeros_v, arr.at[pl.ds(z * ZCHUNK, ZCHUNK)])

    plsc.subcore_barrier()

    def _chunk_body(k, _):
        c = wid + k * NW

        @pl.when(c < N_EDGE_CHUNKS)
        def _():
            pltpu.sync_copy(edges_hbm.at[1, pl.ds(c * CHUNK_ROWS, CHUNK_ROWS)], idx_v)
            for j in range(CHUNK_ROWS):
                pltpu.sync_copy(ones_v, deg_in_s.at[idx_v.at[j]], add=True)
            pltpu.sync_copy(edges_hbm.at[0, pl.ds(c * CHUNK_ROWS, CHUNK_ROWS)], idx_v)
            for j in range(CHUNK_ROWS):
                pltpu.sync_copy(ones_v, deg_out_s.at[idx_v.at[j]], add=True)

        return 0

    lax.fori_loop(0, ROUNDS_A, _chunk_body, 0)

    plsc.subcore_barrier()

    for d, arr in enumerate((deg_in_s, deg_out_s)):
        for k in range(-(-N_ZCHUNKS // NS)):
            z = sid + k * NS

            @pl.when(z < N_ZCHUNKS)
            def _():
                pltpu.sync_copy(arr.at[pl.ds(z * ZCHUNK, ZCHUNK)], zeros_v)
                pltpu.sync_copy(
                    zeros_v,
                    counts_hbm.at[pl.ds((cid * 2 + d) * N_NODES + z * ZCHUNK, ZCHUNK)],
                )


@functools.partial(
    pl.kernel,
    out_type=jax.ShapeDtypeStruct((N_NODES * FEAT,), jnp.float32),
    mesh=_mesh,
    scratch_types=[
        pltpu.VMEM((MAX_DEG * FEAT,), jnp.float32),
        pltpu.VMEM((MAX_DEG * FEAT,), jnp.float32),
        pltpu.VMEM((NODE_CHUNK,), jnp.int32),
        pltpu.VMEM((NODE_CHUNK,), jnp.int32),
        pltpu.VMEM((NODE_CHUNK,), jnp.int32),
        pltpu.VMEM((NODE_CHUNK,), jnp.int32),
        pltpu.VMEM((NODE_CHUNK * FEAT,), jnp.float32),
    ],
    compiler_params=pltpu.CompilerParams(needs_layout_passes=False),
)
def _encode_kernel(counts_hbm, in_tab_hbm, out_tab_hbm, out_hbm,
                   in_tab, out_tab, pin0, pin1, pout0, pout1, outbuf):
    cid = lax.axis_index("c")
    sid = lax.axis_index("s")
    wid = cid * NS + sid

    pltpu.sync_copy(in_tab_hbm, in_tab)
    pltpu.sync_copy(out_tab_hbm, out_tab)

    for k in range(ROUNDS_B):
        c = wid + k * NW

        @pl.when(c < N_NODE_CHUNKS)
        def _():
            base = c * NODE_CHUNK
            pltpu.sync_copy(counts_hbm.at[pl.ds(0 * N_NODES + base, NODE_CHUNK)], pin0)
            pltpu.sync_copy(counts_hbm.at[pl.ds(2 * N_NODES + base, NODE_CHUNK)], pin1)
            pltpu.sync_copy(counts_hbm.at[pl.ds(1 * N_NODES + base, NODE_CHUNK)], pout0)
            pltpu.sync_copy(counts_hbm.at[pl.ds(3 * N_NODES + base, NODE_CHUNK)], pout1)

            def _group_body(g, _):
                off = g * L
                di = jnp.minimum(
                    pin0[pl.ds(off, L)] + pin1[pl.ds(off, L)], MAX_DEG - 1
                ) * FEAT
                do = jnp.minimum(
                    pout0[pl.ds(off, L)] + pout1[pl.ds(off, L)], MAX_DEG - 1
                ) * FEAT
                rows = (lax.iota(jnp.int32, L) + off) * FEAT
                for f in range(FEAT):
                    vi = plsc.load_gather(in_tab, [di + f])
                    vo = plsc.load_gather(out_tab, [do + f])
                    plsc.store_scatter(outbuf, [rows + f], vi + vo)
                return 0

            lax.fori_loop(0, GROUPS, _group_body, 0)
            pltpu.sync_copy(outbuf, out_hbm.at[pl.ds(base * FEAT, NODE_CHUNK * FEAT)])


def kernel(in_embed, out_embed, edge_index_list):
    edges = edge_index_list.astype(jnp.int32).reshape(2, N_EDGES // IDX_W, IDX_W)
    counts = _count_kernel(edges)
    flat = _encode_kernel(
        counts, in_embed.reshape(MAX_DEG * FEAT), out_embed.reshape(MAX_DEG * FEAT)
    )
    return flat.reshape(N_NODES, FEAT)
